# baseline (speedup 1.0000x reference)
import jax
import jax.numpy as jnp
from jax import lax
from jax.experimental import pallas as pl
from jax.experimental.pallas import tpu as pltpu

_N_BINS = 1000
_N_CLS = 91
_K = 100
_Q = 900
_QPAD = 1024
_ROWS = 640
_ER = 104
_B = 8
_NEG = -1e30
_BIGI = 1 << 30


def _topk_kernel(*refs):
    x_refs = refs[:_B]
    box_ref, sc_ref = refs[_B], refs[_B + 1]
    scores_ref, labels_ref, boxes_ref = refs[_B + 2:_B + 5]
    xs_refs = refs[_B + 5:2 * _B + 5]
    ev_refs = refs[2 * _B + 5:3 * _B + 5]
    ei_refs = refs[3 * _B + 5:4 * _B + 5]
    done_ref = refs[4 * _B + 5]

    riota = lax.broadcasted_iota(jnp.int32, (_ROWS, 128), 0)
    ciota = lax.broadcasted_iota(jnp.int32, (1, 128), 1)
    krow = ciota

    for b in range(_B):
        xs_refs[b][...] = x_refs[b][...]
        ev_refs[b][...] = jnp.full((_ER, 128), _NEG, jnp.float32)
        ei_refs[b][...] = jnp.zeros((_ER, 128), jnp.int32)
    done_ref[0] = 0

    def peel(t, _):
        @pl.when(done_ref[0] == 0)
        def _go():
            mremcols = []
            for b in range(_B):
                x = xs_refs[b][...]
                cm = jnp.max(x, axis=0, keepdims=True)
                ar = jnp.min(jnp.where(x == cm, riota, _BIGI),
                             axis=0, keepdims=True)
                ev_refs[b][pl.ds(t, 1), :] = cm
                ei_refs[b][pl.ds(t, 1), :] = ar * 128 + ciota
                xn = jnp.where(riota == ar, _NEG, x)
                xs_refs[b][...] = xn
                mremcols.append(jnp.max(xn, axis=0, keepdims=True))
            mrow = jnp.max(jnp.concatenate(mremcols, axis=0),
                           axis=1, keepdims=True)
            cntcols = []
            for b in range(_B):
                gt = ev_refs[b][...] > mrow[b:b + 1, :]
                cntcols.append(jnp.sum(gt.astype(jnp.int32),
                                       axis=0, keepdims=True))
            crow = jnp.sum(jnp.concatenate(cntcols, axis=0),
                           axis=1, keepdims=True)
            okv = (crow >= _K).astype(jnp.int32)
            allok = jnp.min(okv, axis=0, keepdims=True)
            done_ref[0] = allok[0, 0]
            done_ref[1] = t + 1
        return 0

    lax.fori_loop(0, _K, peel, 0)

    svec0 = tuple(jnp.full((1, 128), _NEG, jnp.float32) for _ in range(_B))
    fvec0 = tuple(jnp.zeros((1, 128), jnp.int32) for _ in range(_B))

    @pl.when(done_ref[1] <= 8)
    def _small_final():
        def body(k, carry):
            evs, eis, svec, fvec = (list(c) for c in carry)
            mrow = jnp.max(
                jnp.concatenate(
                    [jnp.max(evs[b], axis=0, keepdims=True)
                     for b in range(_B)], axis=0),
                axis=1, keepdims=True)
            hits, flcols = [], []
            for b in range(_B):
                hit = evs[b] == mrow[b:b + 1, :]
                hits.append(hit)
                flcols.append(jnp.min(jnp.where(hit, eis[b], _BIGI),
                                      axis=0, keepdims=True))
            frow = jnp.min(jnp.concatenate(flcols, axis=0),
                           axis=1, keepdims=True)
            for b in range(_B):
                fl = frow[b:b + 1, :]
                evs[b] = jnp.where(hits[b] & (eis[b] == fl), _NEG, evs[b])
                svec[b] = jnp.where(krow == k, mrow[b:b + 1, :], svec[b])
                fvec[b] = jnp.where(krow == k, fl, fvec[b])
            return tuple(evs), tuple(eis), tuple(svec), tuple(fvec)

        evs0 = tuple(ev_refs[b][0:8, :] for b in range(_B))
        eis0 = tuple(ei_refs[b][0:8, :] for b in range(_B))
        _, _, svec, fvec = lax.fori_loop(0, _K, body,
                                         (evs0, eis0, svec0, fvec0))
        for b in range(_B):
            scores_ref[pl.ds(b, 1), :] = svec[b]
            labels_ref[pl.ds(b, 1), :] = fvec[b]

    @pl.when(done_ref[1] > 8)
    def _big_final():
        def body(k, carry):
            svec, fvec = list(carry[0]), list(carry[1])
            evl = [ev_refs[b][...] for b in range(_B)]
            eil = [ei_refs[b][...] for b in range(_B)]
            mrow = jnp.max(
                jnp.concatenate(
                    [jnp.max(evl[b], axis=0, keepdims=True)
                     for b in range(_B)], axis=0),
                axis=1, keepdims=True)
            hits, flcols = [], []
            for b in range(_B):
                hit = evl[b] == mrow[b:b + 1, :]
                hits.append(hit)
                flcols.append(jnp.min(jnp.where(hit, eil[b], _BIGI),
                                      axis=0, keepdims=True))
            frow = jnp.min(jnp.concatenate(flcols, axis=0),
                           axis=1, keepdims=True)
            for b in range(_B):
                fl = frow[b:b + 1, :]
                ev_refs[b][...] = jnp.where(hits[b] & (eil[b] == fl),
                                            _NEG, evl[b])
                svec[b] = jnp.where(krow == k, mrow[b:b + 1, :], svec[b])
                fvec[b] = jnp.where(krow == k, fl, fvec[b])
            return tuple(svec), tuple(fvec)

        svec, fvec = lax.fori_loop(0, _K, body, (svec0, fvec0))
        for b in range(_B):
            scores_ref[pl.ds(b, 1), :] = svec[b]
            labels_ref[pl.ds(b, 1), :] = fvec[b]

    qiota = lax.broadcasted_iota(jnp.int32, (_QPAD, 1), 0)
    for b in range(_B):
        sraw = scores_ref[pl.ds(b, 1), :]
        fvb = labels_ref[pl.ds(b, 1), :]
        scores_ref[pl.ds(b, 1), :] = jax.nn.sigmoid(sraw)
        qb = fvb // _N_CLS
        labels_ref[pl.ds(b, 1), :] = fvb - qb * _N_CLS
        bb = box_ref[b, :, :]
        cx, cy, w, h = bb[:, 0:1], bb[:, 1:2], bb[:, 2:3], bb[:, 3:4]
        xy = jnp.concatenate(
            [cx - 0.5 * w, cy - 0.5 * h, cx + 0.5 * w, cy + 0.5 * h], axis=1)
        xy = xy * sc_ref[pl.ds(b, 1), :]
        oht = (qiota == qb).astype(jnp.float32)
        boxes_ref[b, :, :] = lax.dot_general(
            oht, xy, (((0,), (0,)), ((), ())),
            preferred_element_type=jnp.float32)


def kernel(pred_logits, pred_boxes, target_sizes):
    bsz = pred_boxes.shape[0]
    x = pred_logits[4, :, :, _N_BINS:_N_BINS + _N_CLS]
    x = x.reshape(bsz, _Q * _N_CLS)
    x = jnp.pad(x, ((0, 0), (0, _ROWS * 128 - _Q * _N_CLS)),
                constant_values=_NEG)
    x = x.reshape(bsz, _ROWS, 128)
    boxes_pad = jnp.pad(pred_boxes, ((0, 0), (0, _QPAD - _Q), (0, 0)))
    img_h = target_sizes[:, 0]
    img_w = target_sizes[:, 1]
    scale = jnp.stack([img_w, img_h, img_w, img_h], axis=1)

    scores, labels, boxes = pl.pallas_call(
        _topk_kernel,
        out_shape=[
            jax.ShapeDtypeStruct((bsz, 128), jnp.float32),
            jax.ShapeDtypeStruct((bsz, 128), jnp.int32),
            jax.ShapeDtypeStruct((bsz, 128, 4), jnp.float32),
        ],
        scratch_shapes=(
            [pltpu.VMEM((_ROWS, 128), jnp.float32) for _ in range(bsz)]
            + [pltpu.VMEM((_ER, 128), jnp.float32) for _ in range(bsz)]
            + [pltpu.VMEM((_ER, 128), jnp.int32) for _ in range(bsz)]
            + [pltpu.SMEM((2,), jnp.int32)]
        ),
    )(*[x[b] for b in range(bsz)], boxes_pad, scale)
    return scores[:, :_K], labels[:, :_K], boxes[:, :_K, :]


# speedup vs baseline: 1.0159x; 1.0159x over previous
import jax
import jax.numpy as jnp
from jax import lax
from jax.experimental import pallas as pl
from jax.experimental.pallas import tpu as pltpu

_N_BINS = 1000
_N_CLS = 91
_K = 100
_Q = 900
_QPAD = 1024
_ROWS = 640
_ER = 104
_B = 8
_NEG = -1e30
_BIGI = 1 << 30


def _topk_kernel(*refs):
    x_refs = refs[:_B]
    box_ref, sc_ref = refs[_B], refs[_B + 1]
    scores_ref, labels_ref, boxes_ref = refs[_B + 2:_B + 5]
    xs_refs = refs[_B + 5:2 * _B + 5]
    ev_refs = refs[2 * _B + 5:3 * _B + 5]
    ei_refs = refs[3 * _B + 5:4 * _B + 5]
    done_ref = refs[4 * _B + 5]

    riota = lax.broadcasted_iota(jnp.int32, (_ROWS, 128), 0)
    ciota = lax.broadcasted_iota(jnp.int32, (1, 128), 1)
    krow = ciota

    for b in range(_B):
        xs_refs[b][...] = x_refs[b][...]
        ev_refs[b][...] = jnp.full((_ER, 128), _NEG, jnp.float32)
        ei_refs[b][...] = jnp.zeros((_ER, 128), jnp.int32)
    done_ref[0] = 0

    def peel(t, _):
        @pl.when(done_ref[0] == 0)
        def _go():
            flags = None
            for b in range(_B):
                x = xs_refs[b][...]
                cm = jnp.max(x, axis=0, keepdims=True)
                ar = jnp.min(jnp.where(x == cm, riota, _BIGI),
                             axis=0, keepdims=True)
                ev_refs[b][pl.ds(t, 1), :] = cm
                ei_refs[b][pl.ds(t, 1), :] = ar * 128 + ciota
                xn = jnp.where(riota == ar, _NEG, x)
                xs_refs[b][...] = xn
                mrem = jnp.max(xn, axis=0, keepdims=True)
                mrem = jnp.max(mrem, axis=1, keepdims=True)
                cnt = jnp.sum((ev_refs[b][...] > mrem).astype(jnp.int32),
                              axis=0, keepdims=True)
                cnt = jnp.sum(cnt, axis=1, keepdims=True)
                ok = (cnt >= _K).astype(jnp.int32)
                flags = ok if flags is None else flags * ok
            done_ref[0] = flags[0, 0]
            done_ref[1] = t + 1
        return 0

    lax.fori_loop(0, _K, peel, 0)

    svec0 = tuple(jnp.full((1, 128), _NEG, jnp.float32) for _ in range(_B))
    fvec0 = tuple(jnp.zeros((1, 128), jnp.int32) for _ in range(_B))

    @pl.when(done_ref[1] <= 8)
    def _small_final():
        def body(k, carry):
            evs, eis, svec, fvec = (list(c) for c in carry)
            for b in range(_B):
                ev, ei = evs[b], eis[b]
                m = jnp.max(ev, axis=0, keepdims=True)
                m = jnp.max(m, axis=1, keepdims=True)
                hit = ev == m
                fl = jnp.min(jnp.where(hit, ei, _BIGI),
                             axis=0, keepdims=True)
                fl = jnp.min(fl, axis=1, keepdims=True)
                evs[b] = jnp.where(hit & (ei == fl), _NEG, ev)
                svec[b] = jnp.where(krow == k, m, svec[b])
                fvec[b] = jnp.where(krow == k, fl, fvec[b])
            return tuple(evs), tuple(eis), tuple(svec), tuple(fvec)

        evs0 = tuple(ev_refs[b][0:8, :] for b in range(_B))
        eis0 = tuple(ei_refs[b][0:8, :] for b in range(_B))
        _, _, svec, fvec = lax.fori_loop(0, _K, body,
                                         (evs0, eis0, svec0, fvec0))
        for b in range(_B):
            scores_ref[pl.ds(b, 1), :] = svec[b]
            labels_ref[pl.ds(b, 1), :] = fvec[b]

    @pl.when(done_ref[1] > 8)
    def _big_final():
        def body(k, carry):
            svec, fvec = list(carry[0]), list(carry[1])
            for b in range(_B):
                ev = ev_refs[b][...]
                ei = ei_refs[b][...]
                m = jnp.max(ev, axis=0, keepdims=True)
                m = jnp.max(m, axis=1, keepdims=True)
                hit = ev == m
                fl = jnp.min(jnp.where(hit, ei, _BIGI),
                             axis=0, keepdims=True)
                fl = jnp.min(fl, axis=1, keepdims=True)
                ev_refs[b][...] = jnp.where(hit & (ei == fl), _NEG, ev)
                svec[b] = jnp.where(krow == k, m, svec[b])
                fvec[b] = jnp.where(krow == k, fl, fvec[b])
            return tuple(svec), tuple(fvec)

        svec, fvec = lax.fori_loop(0, _K, body, (svec0, fvec0))
        for b in range(_B):
            scores_ref[pl.ds(b, 1), :] = svec[b]
            labels_ref[pl.ds(b, 1), :] = fvec[b]

    qiota = lax.broadcasted_iota(jnp.int32, (_QPAD, 1), 0)
    for b in range(_B):
        sraw = scores_ref[pl.ds(b, 1), :]
        fvb = labels_ref[pl.ds(b, 1), :]
        scores_ref[pl.ds(b, 1), :] = jax.nn.sigmoid(sraw)
        qb = fvb // _N_CLS
        labels_ref[pl.ds(b, 1), :] = fvb - qb * _N_CLS
        bb = box_ref[b, :, :]
        cx, cy, w, h = bb[:, 0:1], bb[:, 1:2], bb[:, 2:3], bb[:, 3:4]
        xy = jnp.concatenate(
            [cx - 0.5 * w, cy - 0.5 * h, cx + 0.5 * w, cy + 0.5 * h], axis=1)
        xy = xy * sc_ref[pl.ds(b, 1), :]
        oht = (qiota == qb).astype(jnp.float32)
        boxes_ref[b, :, :] = lax.dot_general(
            oht, xy, (((0,), (0,)), ((), ())),
            preferred_element_type=jnp.float32)


def kernel(pred_logits, pred_boxes, target_sizes):
    bsz = pred_boxes.shape[0]
    x = pred_logits[4, :, :, _N_BINS:_N_BINS + _N_CLS]
    x = x.reshape(bsz, _Q * _N_CLS)
    x = jnp.pad(x, ((0, 0), (0, _ROWS * 128 - _Q * _N_CLS)),
                constant_values=_NEG)
    x = x.reshape(bsz, _ROWS, 128)
    boxes_pad = jnp.pad(pred_boxes, ((0, 0), (0, _QPAD - _Q), (0, 0)))
    img_h = target_sizes[:, 0]
    img_w = target_sizes[:, 1]
    scale = jnp.stack([img_w, img_h, img_w, img_h], axis=1)

    scores, labels, boxes = pl.pallas_call(
        _topk_kernel,
        out_shape=[
            jax.ShapeDtypeStruct((bsz, 128), jnp.float32),
            jax.ShapeDtypeStruct((bsz, 128), jnp.int32),
            jax.ShapeDtypeStruct((bsz, 128, 4), jnp.float32),
        ],
        scratch_shapes=(
            [pltpu.VMEM((_ROWS, 128), jnp.float32) for _ in range(bsz)]
            + [pltpu.VMEM((_ER, 128), jnp.float32) for _ in range(bsz)]
            + [pltpu.VMEM((_ER, 128), jnp.int32) for _ in range(bsz)]
            + [pltpu.SMEM((2,), jnp.int32)]
        ),
    )(*[x[b] for b in range(bsz)], boxes_pad, scale)
    return scores[:, :_K], labels[:, :_K], boxes[:, :_K, :]
